# baseline (speedup 1.0000x reference)
import functools

import jax
import jax.numpy as jnp
from jax import lax
from jax.experimental import pallas as pl
from jax.experimental.pallas import tpu as pltpu
from jax.experimental.pallas import tpu_sc as plsc

_THRESHOLD = 0.35
_V0, _V1 = 0.1, 0.2
_NEG_RATIO = 3
_T = 16
_BG = 8


def _match_kernel(pcx_ref, pcy_ref, pw_ref, ph_ref,
                  tx0_ref, ty0_ref, tx1_ref, ty1_ref,
                  l0_ref, l1_ref, l2_ref, l3_ref,
                  c0_ref, c1_ref,
                  v_ref, scal_ref, np_ref):
    G, P = l0_ref.shape

    pcx = pcx_ref[...]
    pcy = pcy_ref[...]
    pw = pw_ref[...]
    ph = ph_ref[...]
    px0 = pcx - pw / 2.0
    py0 = pcy - ph / 2.0
    px1 = pcx + pw / 2.0
    py1 = pcy + ph / 2.0
    area_p = (px1 - px0) * (py1 - py0)

    iota = lax.broadcasted_iota(jnp.int32, (1, P), 1)

    tx0a = tx0_ref[...]
    ty0a = ty0_ref[...]
    tx1a = tx1_ref[...]
    ty1a = ty1_ref[...]

    bv = jnp.full((G, P), -1.0, dtype=jnp.float32)
    tm_x0 = jnp.zeros((G, P), dtype=jnp.float32)
    tm_y0 = jnp.zeros((G, P), dtype=jnp.float32)
    tm_x1 = jnp.zeros((G, P), dtype=jnp.float32)
    tm_y1 = jnp.zeros((G, P), dtype=jnp.float32)
    bp_idx = []

    for t in range(_T):
        tx0 = tx0a[:, t:t + 1]
        ty0 = ty0a[:, t:t + 1]
        tx1 = tx1a[:, t:t + 1]
        ty1 = ty1a[:, t:t + 1]
        iw = jnp.clip(jnp.minimum(tx1, px1) - jnp.maximum(tx0, px0), 0.0, None)
        ih = jnp.clip(jnp.minimum(ty1, py1) - jnp.maximum(ty0, py0), 0.0, None)
        inter = iw * ih
        area_t = (tx1 - tx0) * (ty1 - ty0)
        union = area_t + area_p - inter
        iou = inter / union
        m_t = jnp.max(iou, axis=1, keepdims=True)
        idx_t = jnp.min(jnp.where(iou == m_t, iota, P), axis=1, keepdims=True)
        bp_idx.append(idx_t)
        upd = iou > bv
        bv = jnp.where(upd, iou, bv)
        tm_x0 = jnp.where(upd, tx0, tm_x0)
        tm_y0 = jnp.where(upd, ty0, tm_y0)
        tm_x1 = jnp.where(upd, tx1, tm_x1)
        tm_y1 = jnp.where(upd, ty1, tm_y1)

    for t in range(_T):
        mask = iota == bp_idx[t]
        bv = jnp.where(mask, 2.0, bv)
        tm_x0 = jnp.where(mask, tx0a[:, t:t + 1], tm_x0)
        tm_y0 = jnp.where(mask, ty0a[:, t:t + 1], tm_y0)
        tm_x1 = jnp.where(mask, tx1a[:, t:t + 1], tm_x1)
        tm_y1 = jnp.where(mask, ty1a[:, t:t + 1], tm_y1)

    pos = bv >= _THRESHOLD
    np_f = jnp.sum(pos.astype(jnp.float32), axis=1, keepdims=True)

    gx = ((tm_x0 + tm_x1) / 2.0 - pcx) / (_V0 * pw)
    gy = ((tm_y0 + tm_y1) / 2.0 - pcy) / (_V0 * ph)
    gw = jnp.log((tm_x1 - tm_x0) / pw) / _V1
    gh = jnp.log((tm_y1 - tm_y0) / ph) / _V1
    sl1 = jnp.zeros((G, P), dtype=jnp.float32)
    for l_ref, g in ((l0_ref, gx), (l1_ref, gy), (l2_ref, gw), (l3_ref, gh)):
        d = l_ref[...] - g
        a = jnp.abs(d)
        sl1 = sl1 + jnp.where(a < 1.0, 0.5 * d * d, a - 0.5)
    sl1_sum = jnp.sum(jnp.where(pos, sl1, 0.0), axis=1, keepdims=True)

    dm = c1_ref[...] - c0_ref[...]
    lg = jnp.log(1.0 + jnp.exp(-jnp.abs(dm)))
    ce_pos = jnp.maximum(-dm, 0.0) + lg
    v_neg = jnp.maximum(dm, 0.0) + lg
    ce_pos_sum = jnp.sum(jnp.where(pos, ce_pos, 0.0), axis=1, keepdims=True)
    v_ref[...] = jnp.where(pos, 0.0, v_neg)
    scal_ref[...] = jnp.concatenate([np_f, sl1_sum, ce_pos_sum], axis=1)
    np_ref[...] = jnp.broadcast_to(np_f, (G, 16))


def _sc_mine_body(v_hbm, np_hbm, out_hbm, chunk, hcnt, hsum, npv, res):
    c = lax.axis_index("c")
    s = lax.axis_index("s")
    P = 32768

    @pl.when(s < 8)
    def _active():
        row = c * 8 + s
        pltpu.sync_copy(v_hbm.at[row], chunk)
        pltpu.sync_copy(np_hbm.at[row], npv)
        np_s = jnp.max(npv[...])
        k1 = jnp.minimum(_NEG_RATIO * np_s, float(P - 1))

        lane = lax.broadcasted_iota(jnp.int32, (16,), 0)
        zeros16 = jnp.zeros((16,), jnp.float32)
        ones16 = jnp.ones((16,), jnp.float32)

        def zero_hists(nb):
            def zb(i, carry):
                hcnt[pl.ds(i * 16, 16)] = zeros16
                hsum[pl.ds(i * 16, 16)] = zeros16
                return carry
            lax.fori_loop(0, nb // 16, zb, 0)

        def scan_level(nb, k_rem):
            def sb(i, carry):
                found, beta, acnt, asum, ccnt, csum = carry
                cidx = nb // 16 - 1 - i
                vc = hcnt[pl.ds(cidx * 16, 16)]
                vs = hsum[pl.ds(cidx * 16, 16)]
                sufc = lax.rev(jnp.cumsum(lax.rev(vc, (0,))), (0,)) + ccnt
                sufs = lax.rev(jnp.cumsum(lax.rev(vs, (0,))), (0,)) + csum
                m = jnp.max(jnp.where(sufc >= k_rem, lane, -1))
                hit = jnp.logical_and(jnp.logical_not(found), m >= 0)
                lsel = lane == m
                beta = jnp.where(hit, cidx * 16 + m, beta)
                acnt = jnp.where(hit, jnp.sum(jnp.where(lsel, sufc - vc, zeros16)), acnt)
                asum = jnp.where(hit, jnp.sum(jnp.where(lsel, sufs - vs, zeros16)), asum)
                ccnt = ccnt + jnp.sum(vc)
                csum = csum + jnp.sum(vs)
                return (jnp.logical_or(found, hit), beta, acnt, asum, ccnt, csum)
            init = (False, jnp.int32(0), 0.0, 0.0, 0.0, 0.0)
            out = lax.fori_loop(0, nb // 16, sb, init)
            return out[1], out[2], out[3]

        zero_hists(2048)

        def h1(j, carry):
            x = chunk[pl.ds(j * 16, 16)]
            b = lax.shift_right_logical(plsc.bitcast(x, jnp.int32), 20)
            plsc.addupdate_scatter(hcnt, [b], ones16)
            plsc.addupdate_scatter(hsum, [b], x)
            return carry
        lax.fori_loop(0, P // 16, h1, 0)
        beta1, a1, s1 = scan_level(2048, k1)
        k2 = k1 - a1

        zero_hists(4096)

        def h2(j, carry):
            x = chunk[pl.ds(j * 16, 16)]
            bits = plsc.bitcast(x, jnp.int32)
            sub = jnp.bitwise_and(lax.shift_right_logical(bits, 8), 0xFFF)
            msk = lax.shift_right_logical(bits, 20) == beta1
            plsc.addupdate_scatter(hcnt, [sub], ones16, mask=msk)
            plsc.addupdate_scatter(hsum, [sub], x, mask=msk)
            return carry
        lax.fori_loop(0, P // 16, h2, 0)
        beta2, a2, s2 = scan_level(4096, k2)
        k3 = k2 - a2

        zero_hists(256)
        beta12 = jnp.bitwise_or(lax.shift_left(beta1, 12), beta2)

        def h3(j, carry):
            x = chunk[pl.ds(j * 16, 16)]
            bits = plsc.bitcast(x, jnp.int32)
            byt = jnp.bitwise_and(bits, 0xFF)
            msk = lax.shift_right_logical(bits, 8) == beta12
            plsc.addupdate_scatter(hcnt, [byt], ones16, mask=msk)
            plsc.addupdate_scatter(hsum, [byt], x, mask=msk)
            return carry
        lax.fori_loop(0, P // 16, h3, 0)
        beta3, a3, s3 = scan_level(256, k3)

        t_bits = jnp.bitwise_or(lax.shift_left(beta12, 8), beta3)
        t_val = jnp.max(plsc.bitcast(jnp.broadcast_to(t_bits, (16,)), jnp.float32))
        ca = a1 + a2 + a3
        neg_sum = s1 + s2 + s3 + (k1 - ca) * t_val

        res[...] = (jnp.where(lane == 0, neg_sum, 0.0)
                    + jnp.where(lane == 1, ca, 0.0)
                    + jnp.where(lane == 2, t_val, 0.0))
        pltpu.sync_copy(res, out_hbm.at[row])


def _sc_mine(v, np_pad):
    B = v.shape[0]
    f = pl.kernel(
        _sc_mine_body,
        out_type=jax.ShapeDtypeStruct((B, 16), jnp.float32),
        mesh=plsc.VectorSubcoreMesh(core_axis_name="c", subcore_axis_name="s"),
        scratch_types=[
            pltpu.VMEM((32768,), jnp.float32),
            pltpu.VMEM((4096,), jnp.float32),
            pltpu.VMEM((4096,), jnp.float32),
            pltpu.VMEM((16,), jnp.float32),
            pltpu.VMEM((16,), jnp.float32),
        ],
        compiler_params=pltpu.CompilerParams(needs_layout_passes=False),
    )
    return f(v, np_pad)


def _combine_kernel(scal_ref, sc_ref, out_ref):
    P = 32768
    scal = scal_ref[...]
    sc = sc_ref[...]
    np_f = scal[:, 0:1]
    k = jnp.minimum(_NEG_RATIO * np_f, float(P - 1))
    neg_sum = sc[:, 0:1]
    ca = sc[:, 1:2]
    t_val = sc[:, 2:3]
    sel_cnt = np_f + jnp.where(t_val > 0.0, k, ca)
    n = jnp.sum(np_f)
    out_ref[0] = jnp.sum(scal[:, 1:2]) / n
    out_ref[1] = ((jnp.sum(scal[:, 2:3]) + jnp.sum(neg_sum))
                  / jnp.sum(sel_cnt)) / n


def _run(l0, l1, l2, l3, c0, c1, pcx, pcy, pw, ph, tx0, ty0, tx1, ty1,
         interpret=False):
    B, P = l0.shape
    row = lambda b: (b, 0)
    fixed = lambda b: (0, 0)
    rspec = pl.BlockSpec((_BG, P), row)
    tspec = pl.BlockSpec((_BG, _T), row)
    v, scal, np_pad = pl.pallas_call(
        _match_kernel,
        grid=(B // _BG,),
        in_specs=[
            pl.BlockSpec((1, P), fixed), pl.BlockSpec((1, P), fixed),
            pl.BlockSpec((1, P), fixed), pl.BlockSpec((1, P), fixed),
            tspec, tspec, tspec, tspec,
            rspec, rspec, rspec, rspec, rspec, rspec,
        ],
        out_specs=[rspec, pl.BlockSpec((_BG, 3), row),
                   pl.BlockSpec((_BG, 16), row)],
        out_shape=[jax.ShapeDtypeStruct((B, P), jnp.float32),
                   jax.ShapeDtypeStruct((B, 3), jnp.float32),
                   jax.ShapeDtypeStruct((B, 16), jnp.float32)],
        interpret=interpret,
    )(pcx, pcy, pw, ph, tx0, ty0, tx1, ty1, l0, l1, l2, l3, c0, c1)

    sc_out = _sc_mine(v, np_pad)

    return pl.pallas_call(
        _combine_kernel,
        in_specs=[pl.BlockSpec(None), pl.BlockSpec(None)],
        out_specs=pl.BlockSpec(memory_space=pltpu.SMEM),
        out_shape=jax.ShapeDtypeStruct((2,), jnp.float32),
        interpret=interpret,
    )(scal, sc_out)


def kernel(loc_data, conf_data, priors, targets):
    B, P, _ = loc_data.shape
    l0, l1, l2, l3 = (loc_data[:, :, i] for i in range(4))
    c0 = conf_data[:, :, 0]
    c1 = conf_data[:, :, 1]
    pcx = priors[:, 0].reshape(1, P)
    pcy = priors[:, 1].reshape(1, P)
    pw = priors[:, 2].reshape(1, P)
    ph = priors[:, 3].reshape(1, P)
    tx0 = targets[:, :, 0]
    ty0 = targets[:, :, 1]
    tx1 = targets[:, :, 2]
    ty1 = targets[:, :, 3]
    out = _run(l0, l1, l2, l3, c0, c1, pcx, pcy, pw, ph, tx0, ty0, tx1, ty1)
    return out[0], out[1]
